# baseline (speedup 1.0000x reference)
import jax
import jax.numpy as jnp
from jax.experimental import pallas as pl
from jax.experimental.pallas import tpu as pltpu

_BM = 512
_NCHUNK = 8


def _mm_kernel(x_hbm, w_hbm, b_ref, o_ref, xbuf, w_vmem, sem_w, sem_x0, sem_x):
    i = pl.program_id(0)
    nsteps = pl.num_programs(0)
    K = w_vmem.shape[0]
    ck = K // _NCHUNK

    def x0_chunk_copy(c):
        return pltpu.make_async_copy(
            x_hbm.at[pl.ds(0, _BM), pl.ds(c * ck, ck)],
            xbuf.at[0, :, pl.ds(c * ck, ck)],
            sem_x0.at[c],
        )

    def w_chunk_copy(c):
        return pltpu.make_async_copy(
            w_hbm.at[pl.ds(c * ck, ck), :],
            w_vmem.at[pl.ds(c * ck, ck), :],
            sem_w.at[c],
        )

    def x_block_copy(blk, slot):
        return pltpu.make_async_copy(
            x_hbm.at[pl.ds(blk * _BM, _BM), :],
            xbuf.at[slot],
            sem_x.at[slot],
        )

    @pl.when(i == 0)
    def _():
        for c in range(_NCHUNK):
            x0_chunk_copy(c).start()
            w_chunk_copy(c).start()
        x_block_copy(1, 1).start()
        acc = jnp.zeros((_BM, o_ref.shape[1]), jnp.float32) + b_ref[...]
        for c in range(_NCHUNK):
            x0_chunk_copy(c).wait()
            w_chunk_copy(c).wait()
            acc = acc + jnp.dot(
                xbuf[0, :, c * ck : (c + 1) * ck],
                w_vmem[pl.ds(c * ck, ck), :],
                preferred_element_type=jnp.float32,
            )
        o_ref[...] = acc

    @pl.when(i > 0)
    def _():
        slot = jax.lax.rem(i, 2)
        nxt = jax.lax.rem(i + 1, 2)

        @pl.when(i + 1 < nsteps)
        def _():
            x_block_copy(i + 1, nxt).start()

        x_block_copy(i, slot).wait()
        acc = jnp.dot(xbuf[slot], w_vmem[...], preferred_element_type=jnp.float32)
        o_ref[...] = acc + b_ref[...]


def kernel(input, weight, bias):
    M, K = input.shape
    _, N = weight.shape
    bias2d = bias.reshape(1, N)
    return pl.pallas_call(
        _mm_kernel,
        grid=(M // _BM,),
        in_specs=[
            pl.BlockSpec(memory_space=pltpu.MemorySpace.HBM),
            pl.BlockSpec(memory_space=pltpu.MemorySpace.HBM),
            pl.BlockSpec((1, N), lambda i: (0, 0)),
        ],
        out_specs=pl.BlockSpec((_BM, N), lambda i: (i, 0)),
        out_shape=jax.ShapeDtypeStruct((M, N), jnp.float32),
        scratch_shapes=[
            pltpu.VMEM((2, _BM, K), jnp.float32),
            pltpu.VMEM((K, N), jnp.float32),
            pltpu.SemaphoreType.DMA((_NCHUNK,)),
            pltpu.SemaphoreType.DMA((_NCHUNK,)),
            pltpu.SemaphoreType.DMA((2,)),
        ],
    )(input, weight, bias2d)


# speedup vs baseline: 1.0121x; 1.0121x over previous
import jax
import jax.numpy as jnp
from jax.experimental import pallas as pl
from jax.experimental.pallas import tpu as pltpu

_BM = 512
_NCHUNK = 8


def _mm_kernel(x_ref, w_hbm, b_ref, o_ref, w_vmem, sems):
    i = pl.program_id(0)
    K = w_vmem.shape[0]
    ck = K // _NCHUNK

    @pl.when(i == 0)
    def _():
        for c in range(_NCHUNK):
            pltpu.make_async_copy(
                w_hbm.at[pl.ds(c * ck, ck), :],
                w_vmem.at[pl.ds(c * ck, ck), :],
                sems.at[c],
            ).start()
        acc = b_ref[...].astype(jnp.float32)
        for c in range(_NCHUNK):
            pltpu.make_async_copy(
                w_hbm.at[pl.ds(c * ck, ck), :],
                w_vmem.at[pl.ds(c * ck, ck), :],
                sems.at[c],
            ).wait()
            acc = acc + jnp.dot(
                x_ref[:, c * ck : (c + 1) * ck],
                w_vmem[pl.ds(c * ck, ck), :],
                preferred_element_type=jnp.float32,
            )
        o_ref[...] = acc

    @pl.when(i != 0)
    def _():
        acc = jnp.dot(x_ref[...], w_vmem[...], preferred_element_type=jnp.float32)
        o_ref[...] = acc + b_ref[...]


def kernel(input, weight, bias):
    M, K = input.shape
    _, N = weight.shape
    bias2d = bias.reshape(1, N)
    return pl.pallas_call(
        _mm_kernel,
        grid=(M // _BM,),
        in_specs=[
            pl.BlockSpec((_BM, K), lambda i: (i, 0)),
            pl.BlockSpec(memory_space=pltpu.MemorySpace.HBM),
            pl.BlockSpec((1, N), lambda i: (0, 0)),
        ],
        out_specs=pl.BlockSpec((_BM, N), lambda i: (i, 0)),
        out_shape=jax.ShapeDtypeStruct((M, N), jnp.float32),
        scratch_shapes=[
            pltpu.VMEM((K, N), jnp.float32),
            pltpu.SemaphoreType.DMA((_NCHUNK,)),
        ],
    )(input, weight, bias2d)
